# baseline (speedup 1.0000x reference)
import jax
import jax.numpy as jnp
from jax.experimental import pallas as pl

N_BINS = 10
_BR = 1024
_CH = 32


def _h2_err_body(x_ref, lab_ref, h2_ref, err_ref, mn_ref, mx_ref):
    i = pl.program_id(0)
    c = x_ref.shape[1]
    bmin = None
    bmax = None
    for j in range(_BR // _CH):
        x = x_ref[j * _CH:(j + 1) * _CH, :]
        m = jnp.max(x, axis=1, keepdims=True)
        e = jnp.exp(x - m)
        s1 = jnp.sum(e, axis=1, keepdims=True)
        s2 = jnp.sum(e * e, axis=1, keepdims=True)
        h2 = -jnp.log2(s2 / (s1 * s1) + 1e-12)

        col = jax.lax.broadcasted_iota(jnp.int32, (_CH, c), 1)
        cand = jnp.where(x >= m, col, c)
        pred = jnp.min(cand, axis=1, keepdims=True)

        h2_row = jnp.reshape(h2, (1, 1, _CH))
        pred_row = jnp.reshape(pred, (1, 1, _CH))
        lab = lab_ref[:, :, j * _CH:(j + 1) * _CH]
        errf = (pred_row != lab).astype(jnp.float32)

        h2_ref[:, :, j * _CH:(j + 1) * _CH] = h2_row
        err_ref[:, :, j * _CH:(j + 1) * _CH] = errf

        cmin = jnp.min(h2_row, axis=2)
        cmax = jnp.max(h2_row, axis=2)
        bmin = cmin if bmin is None else jnp.minimum(bmin, cmin)
        bmax = cmax if bmax is None else jnp.maximum(bmax, cmax)

    @pl.when(i == 0)
    def _init():
        mn_ref[...] = bmin
        mx_ref[...] = bmax

    @pl.when(i > 0)
    def _acc():
        mn_ref[...] = jnp.minimum(mn_ref[...], bmin)
        mx_ref[...] = jnp.maximum(mx_ref[...], bmax)


def _bin_body(h2_ref, err_ref, mn_ref, mx_ref, uce_ref, errb_ref, h2b_ref):
    h2 = h2_ref[...]
    errf = err_ref[...]
    mn = mn_ref[...]
    mx = mx_ref[...]
    n = h2.size
    step = (mx - mn) / N_BINS
    lane = jax.lax.broadcasted_iota(jnp.int32, (1, N_BINS), 1)
    uce = jnp.zeros((1, 1), jnp.float32)
    errb = jnp.zeros((1, N_BINS), jnp.float32)
    h2b = jnp.zeros((1, N_BINS), jnp.float32)
    for k in range(N_BINS):
        lo = mn + k * step
        hi = mx + 1e-6 if k == N_BINS - 1 else mn + (k + 1) * step
        maskf = ((h2 > lo) & (h2 <= hi)).astype(jnp.float32)
        cnt = jnp.full((1, 1), jnp.sum(maskf))
        safe = jnp.maximum(cnt, 1.0)
        h2_bar = jnp.full((1, 1), jnp.sum(h2 * maskf)) / safe
        err_bar = jnp.full((1, 1), jnp.sum(errf * maskf)) / safe
        inner = jnp.maximum(2.0 * jnp.exp2(-h2_bar) - 1.0, 0.0)
        err_risk = 0.5 * (1.0 - jnp.sqrt(inner))
        nonempty = cnt > 0.0
        prop = cnt / n
        uce = uce + jnp.where(nonempty, jnp.abs(err_bar - err_risk) * prop, 0.0)
        sel = (lane == k).astype(jnp.float32)
        errb = errb + sel * jnp.where(nonempty, err_bar, 0.0)
        h2b = h2b + sel * jnp.where(nonempty, h2_bar, 0.0)
    uce_ref[...] = uce
    errb_ref[...] = errb
    h2b_ref[...] = h2b


def kernel(logits, labels):
    b, c = logits.shape
    nb = b // _BR
    lab3 = labels.reshape(nb, 1, _BR)

    h2, err, mn, mx = pl.pallas_call(
        _h2_err_body,
        grid=(nb,),
        in_specs=[
            pl.BlockSpec((_BR, c), lambda i: (i, 0)),
            pl.BlockSpec((1, 1, _BR), lambda i: (i, 0, 0)),
        ],
        out_specs=[
            pl.BlockSpec((1, 1, _BR), lambda i: (i, 0, 0)),
            pl.BlockSpec((1, 1, _BR), lambda i: (i, 0, 0)),
            pl.BlockSpec((1, 1), lambda i: (0, 0)),
            pl.BlockSpec((1, 1), lambda i: (0, 0)),
        ],
        out_shape=[
            jax.ShapeDtypeStruct((nb, 1, _BR), jnp.float32),
            jax.ShapeDtypeStruct((nb, 1, _BR), jnp.float32),
            jax.ShapeDtypeStruct((1, 1), jnp.float32),
            jax.ShapeDtypeStruct((1, 1), jnp.float32),
        ],
    )(logits, lab3)

    h2r = h2.reshape(nb, _BR)
    errr = err.reshape(nb, _BR)

    uce, errb, h2b = pl.pallas_call(
        _bin_body,
        out_shape=[
            jax.ShapeDtypeStruct((1, 1), jnp.float32),
            jax.ShapeDtypeStruct((1, N_BINS), jnp.float32),
            jax.ShapeDtypeStruct((1, N_BINS), jnp.float32),
        ],
    )(h2r, errr, mn, mx)

    return uce[0, 0], errb.reshape(N_BINS), h2b.reshape(N_BINS)


# speedup vs baseline: 1.5107x; 1.5107x over previous
import jax
import jax.numpy as jnp
from jax.experimental import pallas as pl

N_BINS = 10
_BR = 512


def _h2_err_body(x_ref, lab_ref, h2_ref, err_ref, mn_ref, mx_ref):
    i = pl.program_id(0)
    x = x_ref[...]
    br, c = x.shape
    m = jnp.max(x, axis=1, keepdims=True)
    e = jnp.exp(x - m)
    s1 = jnp.sum(e, axis=1, keepdims=True)
    s2 = jnp.sum(e * e, axis=1, keepdims=True)
    sump2 = s2 / (s1 * s1)
    h2 = -jnp.log2(sump2 + 1e-12)

    col = jax.lax.broadcasted_iota(jnp.int32, (br, c), 1)
    cand = jnp.where(x >= m, col, c)
    pred = jnp.min(cand, axis=1, keepdims=True)

    h2_row = jnp.reshape(h2, (1, br))
    pred_row = jnp.reshape(pred, (1, br))
    lab = lab_ref[0]
    errf = (pred_row != lab).astype(jnp.float32)

    h2_ref[...] = jnp.reshape(h2_row, (1, 1, br))
    err_ref[...] = jnp.reshape(errf, (1, 1, br))

    bmin = jnp.full((1, 1), jnp.min(h2))
    bmax = jnp.full((1, 1), jnp.max(h2))

    @pl.when(i == 0)
    def _init():
        mn_ref[...] = bmin
        mx_ref[...] = bmax

    @pl.when(i > 0)
    def _acc():
        mn_ref[...] = jnp.minimum(mn_ref[...], bmin)
        mx_ref[...] = jnp.maximum(mx_ref[...], bmax)


def _bin_body(h2_ref, err_ref, mn_ref, mx_ref, uce_ref, errb_ref, h2b_ref):
    h2 = h2_ref[...]
    errf = err_ref[...]
    mn = mn_ref[...]
    mx = mx_ref[...]
    n = h2.size
    step = (mx - mn) / N_BINS
    lane = jax.lax.broadcasted_iota(jnp.int32, (1, N_BINS), 1)
    uce = jnp.zeros((1, 1), jnp.float32)
    errb = jnp.zeros((1, N_BINS), jnp.float32)
    h2b = jnp.zeros((1, N_BINS), jnp.float32)
    for k in range(N_BINS):
        lo = mn + k * step
        hi = mx + 1e-6 if k == N_BINS - 1 else mn + (k + 1) * step
        maskf = ((h2 > lo) & (h2 <= hi)).astype(jnp.float32)
        cnt = jnp.full((1, 1), jnp.sum(maskf))
        safe = jnp.maximum(cnt, 1.0)
        h2_bar = jnp.full((1, 1), jnp.sum(h2 * maskf)) / safe
        err_bar = jnp.full((1, 1), jnp.sum(errf * maskf)) / safe
        inner = jnp.maximum(2.0 * jnp.exp2(-h2_bar) - 1.0, 0.0)
        err_risk = 0.5 * (1.0 - jnp.sqrt(inner))
        nonempty = cnt > 0.0
        prop = cnt / n
        uce = uce + jnp.where(nonempty, jnp.abs(err_bar - err_risk) * prop, 0.0)
        sel = (lane == k).astype(jnp.float32)
        errb = errb + sel * jnp.where(nonempty, err_bar, 0.0)
        h2b = h2b + sel * jnp.where(nonempty, h2_bar, 0.0)
    uce_ref[...] = uce
    errb_ref[...] = errb
    h2b_ref[...] = h2b


def kernel(logits, labels):
    b, c = logits.shape
    nb = b // _BR
    lab3 = labels.reshape(nb, 1, _BR)

    h2, err, mn, mx = pl.pallas_call(
        _h2_err_body,
        grid=(nb,),
        in_specs=[
            pl.BlockSpec((_BR, c), lambda i: (i, 0)),
            pl.BlockSpec((1, 1, _BR), lambda i: (i, 0, 0)),
        ],
        out_specs=[
            pl.BlockSpec((1, 1, _BR), lambda i: (i, 0, 0)),
            pl.BlockSpec((1, 1, _BR), lambda i: (i, 0, 0)),
            pl.BlockSpec((1, 1), lambda i: (0, 0)),
            pl.BlockSpec((1, 1), lambda i: (0, 0)),
        ],
        out_shape=[
            jax.ShapeDtypeStruct((nb, 1, _BR), jnp.float32),
            jax.ShapeDtypeStruct((nb, 1, _BR), jnp.float32),
            jax.ShapeDtypeStruct((1, 1), jnp.float32),
            jax.ShapeDtypeStruct((1, 1), jnp.float32),
        ],
    )(logits, lab3)

    h2r = h2.reshape(nb, _BR)
    errr = err.reshape(nb, _BR)

    uce, errb, h2b = pl.pallas_call(
        _bin_body,
        out_shape=[
            jax.ShapeDtypeStruct((1, 1), jnp.float32),
            jax.ShapeDtypeStruct((1, N_BINS), jnp.float32),
            jax.ShapeDtypeStruct((1, N_BINS), jnp.float32),
        ],
    )(h2r, errr, mn, mx)

    return uce[0, 0], errb.reshape(N_BINS), h2b.reshape(N_BINS)


# speedup vs baseline: 1.5774x; 1.0442x over previous
import jax
import jax.numpy as jnp
from jax.experimental import pallas as pl

N_BINS = 10
_BR = 2048


def _h2_err_body(x_ref, lab_ref, h2_ref, err_ref, mn_ref, mx_ref):
    i = pl.program_id(0)
    x = x_ref[...]
    br, c = x.shape
    m = jnp.max(x, axis=1, keepdims=True)
    e = jnp.exp(x - m)
    s1 = jnp.sum(e, axis=1, keepdims=True)
    s2 = jnp.sum(e * e, axis=1, keepdims=True)
    sump2 = s2 / (s1 * s1)
    h2 = -jnp.log2(sump2 + 1e-12)

    col = jax.lax.broadcasted_iota(jnp.int32, (br, c), 1)
    cand = jnp.where(x >= m, col, c)
    pred = jnp.min(cand, axis=1, keepdims=True)

    h2_row = jnp.reshape(h2, (1, br))
    pred_row = jnp.reshape(pred, (1, br))
    lab = lab_ref[0]
    errf = (pred_row != lab).astype(jnp.float32)

    h2_ref[...] = jnp.reshape(h2_row, (1, 1, br))
    err_ref[...] = jnp.reshape(errf, (1, 1, br))

    bmin = jnp.full((1, 1), jnp.min(h2))
    bmax = jnp.full((1, 1), jnp.max(h2))

    @pl.when(i == 0)
    def _init():
        mn_ref[...] = bmin
        mx_ref[...] = bmax

    @pl.when(i > 0)
    def _acc():
        mn_ref[...] = jnp.minimum(mn_ref[...], bmin)
        mx_ref[...] = jnp.maximum(mx_ref[...], bmax)


def _bin_body(h2_ref, err_ref, mn_ref, mx_ref, uce_ref, errb_ref, h2b_ref):
    h2 = h2_ref[...]
    errf = err_ref[...]
    mn = mn_ref[...]
    mx = mx_ref[...]
    n = h2.size
    step = (mx - mn) / N_BINS
    lane = jax.lax.broadcasted_iota(jnp.int32, (1, N_BINS), 1)
    uce = jnp.zeros((1, 1), jnp.float32)
    errb = jnp.zeros((1, N_BINS), jnp.float32)
    h2b = jnp.zeros((1, N_BINS), jnp.float32)
    for k in range(N_BINS):
        lo = mn + k * step
        hi = mx + 1e-6 if k == N_BINS - 1 else mn + (k + 1) * step
        maskf = ((h2 > lo) & (h2 <= hi)).astype(jnp.float32)
        cnt = jnp.full((1, 1), jnp.sum(maskf))
        safe = jnp.maximum(cnt, 1.0)
        h2_bar = jnp.full((1, 1), jnp.sum(h2 * maskf)) / safe
        err_bar = jnp.full((1, 1), jnp.sum(errf * maskf)) / safe
        inner = jnp.maximum(2.0 * jnp.exp2(-h2_bar) - 1.0, 0.0)
        err_risk = 0.5 * (1.0 - jnp.sqrt(inner))
        nonempty = cnt > 0.0
        prop = cnt / n
        uce = uce + jnp.where(nonempty, jnp.abs(err_bar - err_risk) * prop, 0.0)
        sel = (lane == k).astype(jnp.float32)
        errb = errb + sel * jnp.where(nonempty, err_bar, 0.0)
        h2b = h2b + sel * jnp.where(nonempty, h2_bar, 0.0)
    uce_ref[...] = uce
    errb_ref[...] = errb
    h2b_ref[...] = h2b


def kernel(logits, labels):
    b, c = logits.shape
    nb = b // _BR
    lab3 = labels.reshape(nb, 1, _BR)

    h2, err, mn, mx = pl.pallas_call(
        _h2_err_body,
        grid=(nb,),
        in_specs=[
            pl.BlockSpec((_BR, c), lambda i: (i, 0)),
            pl.BlockSpec((1, 1, _BR), lambda i: (i, 0, 0)),
        ],
        out_specs=[
            pl.BlockSpec((1, 1, _BR), lambda i: (i, 0, 0)),
            pl.BlockSpec((1, 1, _BR), lambda i: (i, 0, 0)),
            pl.BlockSpec((1, 1), lambda i: (0, 0)),
            pl.BlockSpec((1, 1), lambda i: (0, 0)),
        ],
        out_shape=[
            jax.ShapeDtypeStruct((nb, 1, _BR), jnp.float32),
            jax.ShapeDtypeStruct((nb, 1, _BR), jnp.float32),
            jax.ShapeDtypeStruct((1, 1), jnp.float32),
            jax.ShapeDtypeStruct((1, 1), jnp.float32),
        ],
    )(logits, lab3)

    h2r = h2.reshape(nb, _BR)
    errr = err.reshape(nb, _BR)

    uce, errb, h2b = pl.pallas_call(
        _bin_body,
        out_shape=[
            jax.ShapeDtypeStruct((1, 1), jnp.float32),
            jax.ShapeDtypeStruct((1, N_BINS), jnp.float32),
            jax.ShapeDtypeStruct((1, N_BINS), jnp.float32),
        ],
    )(h2r, errr, mn, mx)

    return uce[0, 0], errb.reshape(N_BINS), h2b.reshape(N_BINS)
